# baseline (speedup 1.0000x reference)
import functools

import jax
import jax.numpy as jnp
from jax import lax
from jax.experimental import pallas as pl
from jax.experimental.pallas import tpu as pltpu
from jax.experimental.pallas import tpu_sc as plsc

N_NODES = 10000
N_EDGES = 320000
D = 128
D_OUT = 64
EPS = 1e-5

NC = 2
NS = 16
NW = NC * NS
NLOC = 320
DUMMY = NLOC

CHUNK = 16000
NCHUNK = N_EDGES // CHUNK
BATCH = 256
GRP = 8
DEPTH = 2
FLUSH = 1024
RING = 4096
LCAP = 321 * 1024
BIG = 3.4028235e38


def _mesh():
    return plsc.VectorSubcoreMesh(core_axis_name="c", subcore_axis_name="s")


def _wid():
    return lax.axis_index("c") * NS + lax.axis_index("s")


def _bin_kernel(edge_hbm, lists_hbm, cnts_hbm,
                sa0, da0, sa1, da1, sring, dring, cnts_v, sem0, sem1):
    w = _wid()
    lo = w * NLOC
    hi = lo + NLOC
    lanes = lax.iota(jnp.int32, 16)

    def fire_stage(chunk, sv, dv, sem):
        off = chunk * CHUNK
        pltpu.async_copy(edge_hbm.at[0, pl.ds(off, CHUNK)], sv, sem)
        pltpu.async_copy(edge_hbm.at[1, pl.ds(off, CHUNK)], dv, sem)

    def wait_stage(sv, dv, sem):
        pltpu.make_async_copy(edge_hbm.at[0, pl.ds(0, CHUNK)], sv, sem).wait()
        pltpu.make_async_copy(edge_hbm.at[1, pl.ds(0, CHUNK)], dv, sem).wait()

    def scan_chunk(src_v, dst_v, carry):
        def scan_body(i, carry):
            cnt, flushed = carry
            d = dst_v[pl.ds(i * 16, 16)]
            s = src_v[pl.ds(i * 16, 16)]
            m = (d >= lo) & (d < hi)
            npop = plsc.all_reduce_population_count(m)[0]
            posm = cnt & (RING - 1)
            plsc.store_compressed(dring.at[pl.ds(posm, 16)], d - lo, mask=m)
            plsc.store_compressed(sring.at[pl.ds(posm, 16)], s, mask=m)

            @pl.when(posm + npop > RING)
            def _():
                dring[pl.ds(0, 16)] = dring[pl.ds(RING, 16)]
                sring[pl.ds(0, 16)] = sring[pl.ds(RING, 16)]

            cnt = cnt + npop

            def do_flush(carry2):
                cnt, flushed = carry2
                fo = pl.multiple_of(flushed & (RING - 1), FLUSH)
                fh = pl.multiple_of(flushed, FLUSH)
                pltpu.sync_copy(sring.at[pl.ds(fo, FLUSH)],
                                lists_hbm.at[w, 0, pl.ds(fh, FLUSH)])
                pltpu.sync_copy(dring.at[pl.ds(fo, FLUSH)],
                                lists_hbm.at[w, 1, pl.ds(fh, FLUSH)])
                return (cnt, flushed + FLUSH)

            return lax.cond(cnt - flushed >= FLUSH, do_flush,
                            lambda c: c, (cnt, flushed))

        return lax.fori_loop(0, CHUNK // 16, scan_body, carry)

    fire_stage(0, sa0, da0, sem0)
    carry = (jnp.int32(0), jnp.int32(0))

    def two_chunks(t, carry):
        wait_stage(sa0, da0, sem0)
        fire_stage(2 * t + 1, sa1, da1, sem1)
        carry = scan_chunk(sa0, da0, carry)
        wait_stage(sa1, da1, sem1)

        @pl.when(t < NCHUNK // 2 - 1)
        def _():
            fire_stage(2 * t + 2, sa0, da0, sem0)

        return scan_chunk(sa1, da1, carry)

    cnt, flushed = lax.fori_loop(0, NCHUNK // 2, two_chunks, carry)

    for j in range(BATCH // 16):
        pos = (cnt + j * 16 + lanes) & (RING - 1)
        plsc.store_scatter(dring, [pos], jnp.full((16,), DUMMY, jnp.int32))
        plsc.store_scatter(sring, [pos], jnp.zeros((16,), jnp.int32))

    padded = ((cnt + BATCH - 1) // BATCH) * BATCH
    ntail = (padded - flushed) // BATCH

    def tail_body(k, fl):
        fo = pl.multiple_of(fl & (RING - 1), BATCH)
        fh = pl.multiple_of(fl, BATCH)
        pltpu.sync_copy(sring.at[pl.ds(fo, BATCH)],
                        lists_hbm.at[w, 0, pl.ds(fh, BATCH)])
        pltpu.sync_copy(dring.at[pl.ds(fo, BATCH)],
                        lists_hbm.at[w, 1, pl.ds(fh, BATCH)])
        return fl + BATCH

    lax.fori_loop(0, ntail, tail_body, flushed)

    plsc.store_scatter(cnts_v, [jnp.zeros((16,), jnp.int32)],
                       cnt + jnp.zeros((16,), jnp.int32),
                       mask=lanes == 0)
    pltpu.sync_copy(cnts_v, cnts_hbm.at[w])


def _bin_edges(edge_index):
    f = pl.kernel(
        _bin_kernel,
        out_type=(
            jax.ShapeDtypeStruct((NW, 2, LCAP), jnp.int32),
            jax.ShapeDtypeStruct((NW, 16), jnp.int32),
        ),
        mesh=_mesh(),
        compiler_params=pltpu.CompilerParams(needs_layout_passes=False),
        scratch_types=[
            pltpu.VMEM((CHUNK,), jnp.int32),
            pltpu.VMEM((CHUNK,), jnp.int32),
            pltpu.VMEM((CHUNK,), jnp.int32),
            pltpu.VMEM((CHUNK,), jnp.int32),
            pltpu.VMEM((RING + 16,), jnp.int32),
            pltpu.VMEM((RING + 16,), jnp.int32),
            pltpu.VMEM((16,), jnp.int32),
            pltpu.SemaphoreType.DMA,
            pltpu.SemaphoreType.DMA,
        ],
    )
    return f(edge_index)


def _segmin_kernel(h_hbm, lists_hbm, cnts_hbm, agg_hbm,
                   idx0, idx1, ldst0, ldst1, rows0, rows1, cnts_v,
                   psem0, psem1, gsem0, gsem1):
    w = _wid()

    def acc_scope(acc):
        def init_body(r, _):
            for c in range(D // 16):
                acc[r, pl.ds(c * 16, 16)] = jnp.full((16,), BIG, jnp.float32)
            return 0

        lax.fori_loop(0, NLOC + 1, init_body, 0)

        pltpu.sync_copy(cnts_hbm.at[w], cnts_v)
        cnt = cnts_v[pl.ds(0, 16)][0]
        nb = (cnt + BATCH - 1) // BATCH

        def fire_idx(b, idx, ldst, psem):
            base = pl.multiple_of(b * BATCH, BATCH)
            pltpu.async_copy(lists_hbm.at[w, 0, pl.ds(base, BATCH)],
                             idx, psem)
            pltpu.async_copy(lists_hbm.at[w, 1, pl.ds(base, BATCH)],
                             ldst, psem)

        def wait_idx(idx, ldst, psem):
            pltpu.make_async_copy(lists_hbm.at[w, 0, pl.ds(0, BATCH)],
                                  idx, psem).wait()
            pltpu.make_async_copy(lists_hbm.at[w, 1, pl.ds(0, BATCH)],
                                  ldst, psem).wait()

        def fire_gather(idx, rows, gsem):
            pltpu.async_copy(h_hbm.at[idx], rows, gsem)

        def wait_gather(idx, rows, gsem):
            pltpu.make_async_copy(h_hbm.at[idx], rows, gsem).wait()

        def compute(ldst, rows):
            def group_body(eg, _):
                dv = ldst[pl.ds(eg * 16, 16)]
                for j in range(16):
                    dd = dv[j]
                    e = eg * 16 + j
                    for c in range(D // 16):
                        cs = pl.ds(c * 16, 16)
                        acc[dd, cs] = jnp.minimum(acc[dd, cs], rows[e, cs])
                return 0

            lax.fori_loop(0, BATCH // 16, group_body, 0)

        slots = [(idx0, ldst0, rows0, psem0, gsem0),
                 (idx1, ldst1, rows1, psem1, gsem1)]

        fire_idx(0, idx0, ldst0, psem0)

        @pl.when(1 < nb)
        def _():
            fire_idx(1, idx1, ldst1, psem1)

        wait_idx(idx0, ldst0, psem0)
        fire_gather(idx0, rows0, gsem0)

        def pair_body(u, _):
            for s in range(2):
                iA, dA, rA, pA, gA = slots[s]
                iB, dB, rB, pB, gB = slots[1 - s]
                b = 2 * u + s

                @pl.when(b < nb)
                def _(b=b, iA=iA, dA=dA, rA=rA, pA=pA, gA=gA,
                      iB=iB, dB=dB, rB=rB, pB=pB, gB=gB):
                    wait_gather(iA, rA, gA)

                    @pl.when(b + 1 < nb)
                    def _():
                        wait_idx(iB, dB, pB)
                        fire_gather(iB, rB, gB)

                    compute(dA, rA)

                    @pl.when(b + 2 < nb)
                    def _():
                        fire_idx(b + 2, iA, dA, pA)

            return 0

        lax.fori_loop(0, (nb + 1) // 2, pair_body, 0)

        def fix_body(r, _):
            for c in range(D // 16):
                cs = pl.ds(c * 16, 16)
                v = acc[r, cs]
                acc[r, cs] = jnp.where(v > 3e38, jnp.float32(0.0), v)
            return 0

        lax.fori_loop(0, NLOC, fix_body, 0)

        @pl.when(w < NW - 1)
        def _():
            pltpu.sync_copy(acc.at[pl.ds(0, NLOC)],
                            agg_hbm.at[pl.ds(w * NLOC, NLOC)])

        @pl.when(w == NW - 1)
        def _():
            last = N_NODES - (NW - 1) * NLOC
            pltpu.sync_copy(acc.at[pl.ds(0, last)],
                            agg_hbm.at[pl.ds((NW - 1) * NLOC, last)])

    pl.run_scoped(acc_scope, pltpu.VMEM((NLOC + 1, D), jnp.float32))


def _segmin(h, lists, cnts):
    f = pl.kernel(
        _segmin_kernel,
        out_type=jax.ShapeDtypeStruct((N_NODES, D), jnp.float32),
        mesh=_mesh(),
        compiler_params=pltpu.CompilerParams(needs_layout_passes=False),
        scratch_types=[
            pltpu.VMEM((BATCH,), jnp.int32),
            pltpu.VMEM((BATCH,), jnp.int32),
            pltpu.VMEM((BATCH,), jnp.int32),
            pltpu.VMEM((BATCH,), jnp.int32),
            pltpu.VMEM((BATCH, D), jnp.float32),
            pltpu.VMEM((BATCH, D), jnp.float32),
            pltpu.VMEM((16,), jnp.int32),
            pltpu.SemaphoreType.DMA,
            pltpu.SemaphoreType.DMA,
            pltpu.SemaphoreType.DMA,
            pltpu.SemaphoreType.DMA,
        ],
    )
    return f(h, lists, cnts)


ROWS = 1000
GRID = N_NODES // ROWS


def _mm_bn_kernel(agg_ref, x_ref, wl_ref, wr_ref, b_ref, h_ref, sums_ref,
                  acc_ref):
    i = pl.program_id(0)
    h = (jnp.dot(agg_ref[...], wl_ref[...], preferred_element_type=jnp.float32)
         + jnp.dot(x_ref[...], wr_ref[...], preferred_element_type=jnp.float32)
         + b_ref[...])
    h_ref[...] = h

    @pl.when(i == 0)
    def _():
        acc_ref[...] = jnp.zeros_like(acc_ref)

    s1 = jnp.sum(h, axis=0)[None, :]
    s2 = jnp.sum(h * h, axis=0)[None, :]
    acc_ref[0:1, :] += s1
    acc_ref[1:2, :] += s2

    @pl.when(i == GRID - 1)
    def _():
        sums_ref[...] = acc_ref[...]


def _mm_bn(agg, x, wl, wr, b):
    return pl.pallas_call(
        _mm_bn_kernel,
        grid=(GRID,),
        in_specs=[
            pl.BlockSpec((ROWS, D), lambda i: (i, 0)),
            pl.BlockSpec((ROWS, D), lambda i: (i, 0)),
            pl.BlockSpec((D, D), lambda i: (0, 0)),
            pl.BlockSpec((D, D), lambda i: (0, 0)),
            pl.BlockSpec((1, D), lambda i: (0, 0)),
        ],
        out_specs=[
            pl.BlockSpec((ROWS, D), lambda i: (i, 0)),
            pl.BlockSpec((8, D), lambda i: (0, 0)),
        ],
        out_shape=[
            jax.ShapeDtypeStruct((N_NODES, D), jnp.float32),
            jax.ShapeDtypeStruct((8, D), jnp.float32),
        ],
        scratch_shapes=[pltpu.VMEM((8, D), jnp.float32)],
    )(agg, x, wl, wr, b.reshape(1, -1))


def _norm_relu_kernel(h_ref, a_ref, c_ref, o_ref):
    o_ref[...] = jnp.maximum(h_ref[...] * a_ref[...] + c_ref[...], 0.0)


def _norm_relu(h, a, c):
    return pl.pallas_call(
        _norm_relu_kernel,
        grid=(GRID,),
        in_specs=[
            pl.BlockSpec((ROWS, D), lambda i: (i, 0)),
            pl.BlockSpec((1, D), lambda i: (0, 0)),
            pl.BlockSpec((1, D), lambda i: (0, 0)),
        ],
        out_specs=pl.BlockSpec((ROWS, D), lambda i: (i, 0)),
        out_shape=jax.ShapeDtypeStruct((N_NODES, D), jnp.float32),
    )(h, a.reshape(1, -1), c.reshape(1, -1))


def _final_kernel(agg_ref, x_ref, wl_ref, wr_ref, b_ref, o_ref):
    z = (jnp.dot(agg_ref[...], wl_ref[...], preferred_element_type=jnp.float32)
         + jnp.dot(x_ref[...], wr_ref[...], preferred_element_type=jnp.float32)
         + b_ref[...])
    mx = jnp.max(z, axis=1, keepdims=True)
    lse = jnp.log(jnp.sum(jnp.exp(z - mx), axis=1, keepdims=True)) + mx
    o_ref[...] = z - lse


def _final(agg, x, wl, wr, b):
    return pl.pallas_call(
        _final_kernel,
        grid=(GRID,),
        in_specs=[
            pl.BlockSpec((ROWS, D), lambda i: (i, 0)),
            pl.BlockSpec((ROWS, D), lambda i: (i, 0)),
            pl.BlockSpec((D, D_OUT), lambda i: (0, 0)),
            pl.BlockSpec((D, D_OUT), lambda i: (0, 0)),
            pl.BlockSpec((1, D_OUT), lambda i: (0, 0)),
        ],
        out_specs=pl.BlockSpec((ROWS, D_OUT), lambda i: (i, 0)),
        out_shape=jax.ShapeDtypeStruct((N_NODES, D_OUT), jnp.float32),
    )(agg, x, wl, wr, b.reshape(1, -1))


def kernel(x, edge_index, W1l, b1, W1r, g1, be1, W2l, b2, W2r, g2, be2,
           W3l, b3, W3r):
    lists, cnts = _bin_edges(edge_index)

    def sage_bn_layer(h_in, wl, b, wr, g, be):
        agg = _segmin(h_in, lists, cnts)
        h, sums = _mm_bn(agg, h_in, wl, wr, b)
        m = sums[0] / N_NODES
        var = sums[1] / N_NODES - m * m
        a = g / jnp.sqrt(var + EPS)
        c = be - m * a
        return _norm_relu(h, a, c)

    h1 = sage_bn_layer(x, W1l, b1, W1r, g1, be1)
    h2 = sage_bn_layer(h1, W2l, b2, W2r, g2, be2)
    agg3 = _segmin(h2, lists, cnts)
    return _final(agg3, h2, W3l, W3r, b3)


# speedup vs baseline: 1.0633x; 1.0633x over previous
import functools

import jax
import jax.numpy as jnp
from jax import lax
from jax.experimental import pallas as pl
from jax.experimental.pallas import tpu as pltpu
from jax.experimental.pallas import tpu_sc as plsc

N_NODES = 10000
N_EDGES = 320000
D = 128
D_OUT = 64
EPS = 1e-5

NC = 2
NS = 16
NW = NC * NS
NLOC = 320
DUMMY = NLOC

CHUNK = 16000
NCHUNK = N_EDGES // CHUNK
BATCH = 128
FLUSH = 1024
RING = 4096
LCAP = 321 * 1024
BIG = 3.4028235e38


def _mesh():
    return plsc.VectorSubcoreMesh(core_axis_name="c", subcore_axis_name="s")


def _wid():
    return lax.axis_index("c") * NS + lax.axis_index("s")


def _bin_kernel(edge_hbm, lists_hbm, cnts_hbm,
                sa0, da0, sa1, da1, sring, dring, cnts_v, sem0, sem1):
    w = _wid()
    lo = w * NLOC
    hi = lo + NLOC
    lanes = lax.iota(jnp.int32, 16)

    def fire_stage(chunk, sv, dv, sem):
        off = chunk * CHUNK
        pltpu.async_copy(edge_hbm.at[0, pl.ds(off, CHUNK)], sv, sem)
        pltpu.async_copy(edge_hbm.at[1, pl.ds(off, CHUNK)], dv, sem)

    def wait_stage(sv, dv, sem):
        pltpu.make_async_copy(edge_hbm.at[0, pl.ds(0, CHUNK)], sv, sem).wait()
        pltpu.make_async_copy(edge_hbm.at[1, pl.ds(0, CHUNK)], dv, sem).wait()

    def scan_chunk(src_v, dst_v, carry):
        def scan_body(i, carry):
            cnt, flushed = carry
            d = dst_v[pl.ds(i * 16, 16)]
            s = src_v[pl.ds(i * 16, 16)]
            m = (d >= lo) & (d < hi)
            npop = plsc.all_reduce_population_count(m)[0]
            posm = cnt & (RING - 1)
            plsc.store_compressed(dring.at[pl.ds(posm, 16)], d - lo, mask=m)
            plsc.store_compressed(sring.at[pl.ds(posm, 16)], s, mask=m)

            @pl.when(posm + npop > RING)
            def _():
                dring[pl.ds(0, 16)] = dring[pl.ds(RING, 16)]
                sring[pl.ds(0, 16)] = sring[pl.ds(RING, 16)]

            cnt = cnt + npop

            def do_flush(carry2):
                cnt, flushed = carry2
                fo = pl.multiple_of(flushed & (RING - 1), FLUSH)
                fh = pl.multiple_of(flushed, FLUSH)
                pltpu.sync_copy(sring.at[pl.ds(fo, FLUSH)],
                                lists_hbm.at[w, 0, pl.ds(fh, FLUSH)])
                pltpu.sync_copy(dring.at[pl.ds(fo, FLUSH)],
                                lists_hbm.at[w, 1, pl.ds(fh, FLUSH)])
                return (cnt, flushed + FLUSH)

            return lax.cond(cnt - flushed >= FLUSH, do_flush,
                            lambda c: c, (cnt, flushed))

        return lax.fori_loop(0, CHUNK // 16, scan_body, carry)

    fire_stage(0, sa0, da0, sem0)
    carry = (jnp.int32(0), jnp.int32(0))

    def two_chunks(t, carry):
        wait_stage(sa0, da0, sem0)
        fire_stage(2 * t + 1, sa1, da1, sem1)
        carry = scan_chunk(sa0, da0, carry)
        wait_stage(sa1, da1, sem1)

        @pl.when(t < NCHUNK // 2 - 1)
        def _():
            fire_stage(2 * t + 2, sa0, da0, sem0)

        return scan_chunk(sa1, da1, carry)

    cnt, flushed = lax.fori_loop(0, NCHUNK // 2, two_chunks, carry)

    for j in range(BATCH // 16):
        pos = (cnt + j * 16 + lanes) & (RING - 1)
        plsc.store_scatter(dring, [pos], jnp.full((16,), DUMMY, jnp.int32))
        plsc.store_scatter(sring, [pos], jnp.zeros((16,), jnp.int32))

    padded = ((cnt + BATCH - 1) // BATCH) * BATCH
    ntail = (padded - flushed) // BATCH

    def tail_body(k, fl):
        fo = pl.multiple_of(fl & (RING - 1), BATCH)
        fh = pl.multiple_of(fl, BATCH)
        pltpu.sync_copy(sring.at[pl.ds(fo, BATCH)],
                        lists_hbm.at[w, 0, pl.ds(fh, BATCH)])
        pltpu.sync_copy(dring.at[pl.ds(fo, BATCH)],
                        lists_hbm.at[w, 1, pl.ds(fh, BATCH)])
        return fl + BATCH

    lax.fori_loop(0, ntail, tail_body, flushed)

    plsc.store_scatter(cnts_v, [jnp.zeros((16,), jnp.int32)],
                       cnt + jnp.zeros((16,), jnp.int32),
                       mask=lanes == 0)
    pltpu.sync_copy(cnts_v, cnts_hbm.at[w])


def _bin_edges(edge_index):
    f = pl.kernel(
        _bin_kernel,
        out_type=(
            jax.ShapeDtypeStruct((NW, 2, LCAP), jnp.int32),
            jax.ShapeDtypeStruct((NW, 16), jnp.int32),
        ),
        mesh=_mesh(),
        compiler_params=pltpu.CompilerParams(needs_layout_passes=False),
        scratch_types=[
            pltpu.VMEM((CHUNK,), jnp.int32),
            pltpu.VMEM((CHUNK,), jnp.int32),
            pltpu.VMEM((CHUNK,), jnp.int32),
            pltpu.VMEM((CHUNK,), jnp.int32),
            pltpu.VMEM((RING + 16,), jnp.int32),
            pltpu.VMEM((RING + 16,), jnp.int32),
            pltpu.VMEM((16,), jnp.int32),
            pltpu.SemaphoreType.DMA,
            pltpu.SemaphoreType.DMA,
        ],
    )
    return f(edge_index)


def _segmin_kernel(h_hbm, lists_hbm, cnts_hbm, agg_hbm,
                   idx0, idx1, ldst0, ldst1, rows0, rows1, cnts_v,
                   psem0, psem1, gsem0, gsem1):
    w = _wid()

    def acc_scope(acc):
        def init_body(r, _):
            for c in range(D // 16):
                acc[r, pl.ds(c * 16, 16)] = jnp.full((16,), BIG, jnp.float32)
            return 0

        lax.fori_loop(0, NLOC + 1, init_body, 0)

        pltpu.sync_copy(cnts_hbm.at[w], cnts_v)
        cnt = cnts_v[pl.ds(0, 16)][0]
        nb = (cnt + BATCH - 1) // BATCH

        def fire_idx(b, idx, ldst, psem):
            base = pl.multiple_of(b * BATCH, BATCH)
            pltpu.async_copy(lists_hbm.at[w, 0, pl.ds(base, BATCH)],
                             idx, psem)
            pltpu.async_copy(lists_hbm.at[w, 1, pl.ds(base, BATCH)],
                             ldst, psem)

        def wait_idx(idx, ldst, psem):
            pltpu.make_async_copy(lists_hbm.at[w, 0, pl.ds(0, BATCH)],
                                  idx, psem).wait()
            pltpu.make_async_copy(lists_hbm.at[w, 1, pl.ds(0, BATCH)],
                                  ldst, psem).wait()

        def fire_gather(idx, rows, gsem):
            pltpu.async_copy(h_hbm.at[idx], rows, gsem)

        def wait_gather(idx, rows, gsem):
            pltpu.make_async_copy(h_hbm.at[idx], rows, gsem).wait()

        def compute(ldst, rows):
            def group_body(eg, _):
                dv = ldst[pl.ds(eg * 16, 16)]
                for j in range(16):
                    dd = dv[j]
                    e = eg * 16 + j
                    for c in range(D // 16):
                        cs = pl.ds(c * 16, 16)
                        acc[dd, cs] = jnp.minimum(acc[dd, cs], rows[e, cs])
                return 0

            lax.fori_loop(0, BATCH // 16, group_body, 0)

        slots = [(idx0, ldst0, rows0, psem0, gsem0),
                 (idx1, ldst1, rows1, psem1, gsem1)]

        fire_idx(0, idx0, ldst0, psem0)

        @pl.when(1 < nb)
        def _():
            fire_idx(1, idx1, ldst1, psem1)

        wait_idx(idx0, ldst0, psem0)
        fire_gather(idx0, rows0, gsem0)

        def pair_body(u, _):
            for s in range(2):
                iA, dA, rA, pA, gA = slots[s]
                iB, dB, rB, pB, gB = slots[1 - s]
                b = 2 * u + s

                @pl.when(b < nb)
                def _(b=b, iA=iA, dA=dA, rA=rA, pA=pA, gA=gA,
                      iB=iB, dB=dB, rB=rB, pB=pB, gB=gB):
                    wait_gather(iA, rA, gA)

                    @pl.when(b + 1 < nb)
                    def _():
                        wait_idx(iB, dB, pB)
                        fire_gather(iB, rB, gB)

                    compute(dA, rA)

                    @pl.when(b + 2 < nb)
                    def _():
                        fire_idx(b + 2, iA, dA, pA)

            return 0

        lax.fori_loop(0, (nb + 1) // 2, pair_body, 0)

        def fix_body(r, _):
            for c in range(D // 16):
                cs = pl.ds(c * 16, 16)
                v = acc[r, cs]
                acc[r, cs] = jnp.where(v > 3e38, jnp.float32(0.0), v)
            return 0

        lax.fori_loop(0, NLOC, fix_body, 0)

        @pl.when(w < NW - 1)
        def _():
            pltpu.sync_copy(acc.at[pl.ds(0, NLOC)],
                            agg_hbm.at[pl.ds(w * NLOC, NLOC)])

        @pl.when(w == NW - 1)
        def _():
            last = N_NODES - (NW - 1) * NLOC
            pltpu.sync_copy(acc.at[pl.ds(0, last)],
                            agg_hbm.at[pl.ds((NW - 1) * NLOC, last)])

    pl.run_scoped(acc_scope, pltpu.VMEM((NLOC + 1, D), jnp.float32))


def _segmin(h, lists, cnts):
    f = pl.kernel(
        _segmin_kernel,
        out_type=jax.ShapeDtypeStruct((N_NODES, D), jnp.float32),
        mesh=_mesh(),
        compiler_params=pltpu.CompilerParams(needs_layout_passes=False),
        scratch_types=[
            pltpu.VMEM((BATCH,), jnp.int32),
            pltpu.VMEM((BATCH,), jnp.int32),
            pltpu.VMEM((BATCH,), jnp.int32),
            pltpu.VMEM((BATCH,), jnp.int32),
            pltpu.VMEM((BATCH, D), jnp.float32),
            pltpu.VMEM((BATCH, D), jnp.float32),
            pltpu.VMEM((16,), jnp.int32),
            pltpu.SemaphoreType.DMA,
            pltpu.SemaphoreType.DMA,
            pltpu.SemaphoreType.DMA,
            pltpu.SemaphoreType.DMA,
        ],
    )
    return f(h, lists, cnts)


ROWS = 1000
GRID = N_NODES // ROWS


def _mm_bn_kernel(agg_ref, x_ref, wl_ref, wr_ref, b_ref, h_ref, sums_ref,
                  acc_ref):
    i = pl.program_id(0)
    h = (jnp.dot(agg_ref[...], wl_ref[...], preferred_element_type=jnp.float32)
         + jnp.dot(x_ref[...], wr_ref[...], preferred_element_type=jnp.float32)
         + b_ref[...])
    h_ref[...] = h

    @pl.when(i == 0)
    def _():
        acc_ref[...] = jnp.zeros_like(acc_ref)

    s1 = jnp.sum(h, axis=0)[None, :]
    s2 = jnp.sum(h * h, axis=0)[None, :]
    acc_ref[0:1, :] += s1
    acc_ref[1:2, :] += s2

    @pl.when(i == GRID - 1)
    def _():
        sums_ref[...] = acc_ref[...]


def _mm_bn(agg, x, wl, wr, b):
    return pl.pallas_call(
        _mm_bn_kernel,
        grid=(GRID,),
        in_specs=[
            pl.BlockSpec((ROWS, D), lambda i: (i, 0)),
            pl.BlockSpec((ROWS, D), lambda i: (i, 0)),
            pl.BlockSpec((D, D), lambda i: (0, 0)),
            pl.BlockSpec((D, D), lambda i: (0, 0)),
            pl.BlockSpec((1, D), lambda i: (0, 0)),
        ],
        out_specs=[
            pl.BlockSpec((ROWS, D), lambda i: (i, 0)),
            pl.BlockSpec((8, D), lambda i: (0, 0)),
        ],
        out_shape=[
            jax.ShapeDtypeStruct((N_NODES, D), jnp.float32),
            jax.ShapeDtypeStruct((8, D), jnp.float32),
        ],
        scratch_shapes=[pltpu.VMEM((8, D), jnp.float32)],
    )(agg, x, wl, wr, b.reshape(1, -1))


def _norm_relu_kernel(h_ref, a_ref, c_ref, o_ref):
    o_ref[...] = jnp.maximum(h_ref[...] * a_ref[...] + c_ref[...], 0.0)


def _norm_relu(h, a, c):
    return pl.pallas_call(
        _norm_relu_kernel,
        grid=(GRID,),
        in_specs=[
            pl.BlockSpec((ROWS, D), lambda i: (i, 0)),
            pl.BlockSpec((1, D), lambda i: (0, 0)),
            pl.BlockSpec((1, D), lambda i: (0, 0)),
        ],
        out_specs=pl.BlockSpec((ROWS, D), lambda i: (i, 0)),
        out_shape=jax.ShapeDtypeStruct((N_NODES, D), jnp.float32),
    )(h, a.reshape(1, -1), c.reshape(1, -1))


def _final_kernel(agg_ref, x_ref, wl_ref, wr_ref, b_ref, o_ref):
    z = (jnp.dot(agg_ref[...], wl_ref[...], preferred_element_type=jnp.float32)
         + jnp.dot(x_ref[...], wr_ref[...], preferred_element_type=jnp.float32)
         + b_ref[...])
    mx = jnp.max(z, axis=1, keepdims=True)
    lse = jnp.log(jnp.sum(jnp.exp(z - mx), axis=1, keepdims=True)) + mx
    o_ref[...] = z - lse


def _final(agg, x, wl, wr, b):
    return pl.pallas_call(
        _final_kernel,
        grid=(GRID,),
        in_specs=[
            pl.BlockSpec((ROWS, D), lambda i: (i, 0)),
            pl.BlockSpec((ROWS, D), lambda i: (i, 0)),
            pl.BlockSpec((D, D_OUT), lambda i: (0, 0)),
            pl.BlockSpec((D, D_OUT), lambda i: (0, 0)),
            pl.BlockSpec((1, D_OUT), lambda i: (0, 0)),
        ],
        out_specs=pl.BlockSpec((ROWS, D_OUT), lambda i: (i, 0)),
        out_shape=jax.ShapeDtypeStruct((N_NODES, D_OUT), jnp.float32),
    )(agg, x, wl, wr, b.reshape(1, -1))


def kernel(x, edge_index, W1l, b1, W1r, g1, be1, W2l, b2, W2r, g2, be2,
           W3l, b3, W3r):
    lists, cnts = _bin_edges(edge_index)

    def sage_bn_layer(h_in, wl, b, wr, g, be):
        agg = _segmin(h_in, lists, cnts)
        h, sums = _mm_bn(agg, h_in, wl, wr, b)
        m = sums[0] / N_NODES
        var = sums[1] / N_NODES - m * m
        a = g / jnp.sqrt(var + EPS)
        c = be - m * a
        return _norm_relu(h, a, c)

    h1 = sage_bn_layer(x, W1l, b1, W1r, g1, be1)
    h2 = sage_bn_layer(h1, W2l, b2, W2r, g2, be2)
    agg3 = _segmin(h2, lists, cnts)
    return _final(agg3, h2, W3l, W3r, b3)


# speedup vs baseline: 1.0920x; 1.0269x over previous
import functools

import jax
import jax.numpy as jnp
from jax import lax
from jax.experimental import pallas as pl
from jax.experimental.pallas import tpu as pltpu
from jax.experimental.pallas import tpu_sc as plsc

N_NODES = 10000
N_EDGES = 320000
D = 128
D_OUT = 64
EPS = 1e-5

NC = 2
NS = 16
NW = NC * NS
NLOC = 320
DUMMY = NLOC

QN = 4
EQUART = N_EDGES // QN
CHUNK = 16000
NCHUNK = EQUART // CHUNK
BATCH = 128
FLUSH = 1024
RING = 4096
LCAP4 = 81920
LCAP = QN * LCAP4
BIG = 3.4028235e38


def _mesh():
    return plsc.VectorSubcoreMesh(core_axis_name="c", subcore_axis_name="s")


def _wid():
    return lax.axis_index("c") * NS + lax.axis_index("s")


def _bin_kernel(edge_hbm, lists_hbm, cnts_hbm,
                sa0, da0, sa1, da1,
                sr0, dr0, sr1, dr1, sr2, dr2, sr3, dr3,
                cnts_v, sem0, sem1):
    w = _wid()
    g = w // QN
    q = w % QN
    glo = g * QN * NLOC
    eq0 = q * EQUART
    lanes = lax.iota(jnp.int32, 16)
    srings = [sr0, sr1, sr2, sr3]
    drings = [dr0, dr1, dr2, dr3]

    def fire_stage(chunk, sv, dv, sem):
        off = eq0 + chunk * CHUNK
        pltpu.async_copy(edge_hbm.at[0, pl.ds(off, CHUNK)], sv, sem)
        pltpu.async_copy(edge_hbm.at[1, pl.ds(off, CHUNK)], dv, sem)

    def wait_stage(sv, dv, sem):
        pltpu.make_async_copy(edge_hbm.at[0, pl.ds(0, CHUNK)], sv, sem).wait()
        pltpu.make_async_copy(edge_hbm.at[1, pl.ds(0, CHUNK)], dv, sem).wait()

    def flush_one(r, fl):
        fo = pl.multiple_of(fl & (RING - 1), FLUSH)
        fh = pl.multiple_of(fl, FLUSH)
        hb = pl.multiple_of(q * LCAP4 + fh, FLUSH)
        pltpu.sync_copy(srings[r].at[pl.ds(fo, FLUSH)],
                        lists_hbm.at[g * QN + r, 0, pl.ds(hb, FLUSH)])
        pltpu.sync_copy(drings[r].at[pl.ds(fo, FLUSH)],
                        lists_hbm.at[g * QN + r, 1, pl.ds(hb, FLUSH)])
        return fl + FLUSH

    def scan_chunk(src_v, dst_v, carry):
        nsub = CHUNK // 16 // 25

        def scan_step(i, cnts):
            d = dst_v[pl.ds(i * 16, 16)]
            s = src_v[pl.ds(i * 16, 16)]
            local = d - glo
            m = (local >= 0) & (local < QN * NLOC)
            out = []
            for r in range(QN):
                mr = m & (local >= r * NLOC) & (local < (r + 1) * NLOC)
                npop = plsc.all_reduce_population_count(mr)[0]
                posm = cnts[r] & (RING - 1)
                plsc.store_compressed(drings[r].at[pl.ds(posm, 16)],
                                      local - r * NLOC, mask=mr)
                plsc.store_compressed(srings[r].at[pl.ds(posm, 16)],
                                      s, mask=mr)

                @pl.when(posm + npop > RING)
                def _(r=r):
                    drings[r][pl.ds(0, 16)] = drings[r][pl.ds(RING, 16)]
                    srings[r][pl.ds(0, 16)] = srings[r][pl.ds(RING, 16)]

                out.append(cnts[r] + npop)
            return tuple(out)

        def outer_step(o, carry):
            cnts = carry[:QN]
            fls = carry[QN:]
            cnts = lax.fori_loop(o * 25, o * 25 + 25, scan_step, cnts)
            nfls = []
            for r in range(QN):
                fl = lax.cond(cnts[r] - fls[r] >= FLUSH,
                              lambda fl, r=r: flush_one(r, fl),
                              lambda fl: fl, fls[r])
                nfls.append(fl)
            return cnts + tuple(nfls)

        return lax.fori_loop(0, nsub, outer_step, carry)

    fire_stage(0, sa0, da0, sem0)
    carry = tuple(jnp.int32(0) for _ in range(2 * QN))

    def two_chunks(t, carry):
        wait_stage(sa0, da0, sem0)
        fire_stage(2 * t + 1, sa1, da1, sem1)
        carry = scan_chunk(sa0, da0, carry)
        wait_stage(sa1, da1, sem1)

        @pl.when(2 * t + 2 < NCHUNK)
        def _():
            fire_stage(2 * t + 2, sa0, da0, sem0)

        return scan_chunk(sa1, da1, carry)

    carry = lax.fori_loop(0, NCHUNK // 2, two_chunks, carry)
    if NCHUNK % 2:
        wait_stage(sa0, da0, sem0)
        carry = scan_chunk(sa0, da0, carry)

    for r in range(QN):
        cnt = carry[r]
        flushed = carry[QN + r]
        for j in range(BATCH // 16):
            pos = (cnt + j * 16 + lanes) & (RING - 1)
            plsc.store_scatter(drings[r], [pos],
                               jnp.full((16,), DUMMY, jnp.int32))
            plsc.store_scatter(srings[r], [pos], jnp.zeros((16,), jnp.int32))

        padded = ((cnt + BATCH - 1) // BATCH) * BATCH
        ntail = (padded - flushed) // BATCH

        def tail_body(k, fl, r=r):
            fo = pl.multiple_of(fl & (RING - 1), BATCH)
            fh = pl.multiple_of(fl, BATCH)
            hb = pl.multiple_of(q * LCAP4 + fh, BATCH)
            pltpu.sync_copy(srings[r].at[pl.ds(fo, BATCH)],
                            lists_hbm.at[g * QN + r, 0, pl.ds(hb, BATCH)])
            pltpu.sync_copy(drings[r].at[pl.ds(fo, BATCH)],
                            lists_hbm.at[g * QN + r, 1, pl.ds(hb, BATCH)])
            return fl + BATCH

        lax.fori_loop(0, ntail, tail_body, flushed)

        plsc.store_scatter(cnts_v, [jnp.zeros((16,), jnp.int32)],
                           cnt + jnp.zeros((16,), jnp.int32),
                           mask=lanes == 0)
        pltpu.sync_copy(cnts_v, cnts_hbm.at[g * QN + r, q])


def _bin_edges(edge_index):
    f = pl.kernel(
        _bin_kernel,
        out_type=(
            jax.ShapeDtypeStruct((NW, 2, LCAP), jnp.int32),
            jax.ShapeDtypeStruct((NW, QN, 16), jnp.int32),
        ),
        mesh=_mesh(),
        compiler_params=pltpu.CompilerParams(needs_layout_passes=False),
        scratch_types=(
            [pltpu.VMEM((CHUNK,), jnp.int32) for _ in range(4)]
            + [pltpu.VMEM((RING + 16,), jnp.int32) for _ in range(2 * QN)]
            + [pltpu.VMEM((16,), jnp.int32),
               pltpu.SemaphoreType.DMA,
               pltpu.SemaphoreType.DMA]
        ),
    )
    return f(edge_index)


def _segmin_kernel(h_hbm, lists_hbm, cnts_hbm, agg_hbm,
                   idx0, idx1, ldst0, ldst1, rows0, rows1, cnts_v,
                   psem0, psem1, gsem0, gsem1):
    w = _wid()

    def acc_scope(acc):
        def init_body(r, _):
            for c in range(D // 16):
                acc[r, pl.ds(c * 16, 16)] = jnp.full((16,), BIG, jnp.float32)
            return 0

        lax.fori_loop(0, NLOC + 1, init_body, 0)

        pltpu.sync_copy(cnts_hbm.at[w], cnts_v)
        subcnts = [cnts_v[qq, pl.ds(0, 16)][0] for qq in range(QN)]

        def fire_gather(idx, rows, gsem):
            pltpu.async_copy(h_hbm.at[idx], rows, gsem)

        def wait_gather(idx, rows, gsem):
            pltpu.make_async_copy(h_hbm.at[idx], rows, gsem).wait()

        def compute(ldst, rows):
            def group_body(eg, _):
                dv = ldst[pl.ds(eg * 16, 16)]
                for j in range(16):
                    dd = dv[j]
                    e = eg * 16 + j
                    for c in range(D // 16):
                        cs = pl.ds(c * 16, 16)
                        acc[dd, cs] = jnp.minimum(acc[dd, cs], rows[e, cs])
                return 0

            lax.fori_loop(0, BATCH // 16, group_body, 0)

        slots = [(idx0, ldst0, rows0, psem0, gsem0),
                 (idx1, ldst1, rows1, psem1, gsem1)]

        def seg_body(qq, _):
            cnt = subcnts[0]
            for q2 in range(1, QN):
                cnt = jnp.where(qq == q2, subcnts[q2], cnt)
            nb = (cnt + BATCH - 1) // BATCH
            base0 = qq * LCAP4

            def fire_idx(b, idx, ldst, psem):
                base = pl.multiple_of(base0 + b * BATCH, BATCH)
                pltpu.async_copy(lists_hbm.at[w, 0, pl.ds(base, BATCH)],
                                 idx, psem)
                pltpu.async_copy(lists_hbm.at[w, 1, pl.ds(base, BATCH)],
                                 ldst, psem)

            def wait_idx(idx, ldst, psem):
                pltpu.make_async_copy(lists_hbm.at[w, 0, pl.ds(0, BATCH)],
                                      idx, psem).wait()
                pltpu.make_async_copy(lists_hbm.at[w, 1, pl.ds(0, BATCH)],
                                      ldst, psem).wait()

            @pl.when(nb > 0)
            def _():
                fire_idx(0, idx0, ldst0, psem0)

                @pl.when(1 < nb)
                def _():
                    fire_idx(1, idx1, ldst1, psem1)

                wait_idx(idx0, ldst0, psem0)
                fire_gather(idx0, rows0, gsem0)

                def pair_body(u, _):
                    for s in range(2):
                        iA, dA, rA, pA, gA = slots[s]
                        iB, dB, rB, pB, gB = slots[1 - s]
                        b = 2 * u + s

                        @pl.when(b < nb)
                        def _(b=b, iA=iA, dA=dA, rA=rA, pA=pA, gA=gA,
                              iB=iB, dB=dB, rB=rB, pB=pB, gB=gB):
                            wait_gather(iA, rA, gA)

                            @pl.when(b + 1 < nb)
                            def _():
                                wait_idx(iB, dB, pB)
                                fire_gather(iB, rB, gB)

                            compute(dA, rA)

                            @pl.when(b + 2 < nb)
                            def _():
                                fire_idx(b + 2, iA, dA, pA)

                    return 0

                lax.fori_loop(0, (nb + 1) // 2, pair_body, 0)

            return 0

        lax.fori_loop(0, QN, seg_body, 0)

        def fix_body(r, _):
            for c in range(D // 16):
                cs = pl.ds(c * 16, 16)
                v = acc[r, cs]
                acc[r, cs] = jnp.where(v > 3e38, jnp.float32(0.0), v)
            return 0

        lax.fori_loop(0, NLOC, fix_body, 0)

        @pl.when(w < NW - 1)
        def _():
            pltpu.sync_copy(acc.at[pl.ds(0, NLOC)],
                            agg_hbm.at[pl.ds(w * NLOC, NLOC)])

        @pl.when(w == NW - 1)
        def _():
            last = N_NODES - (NW - 1) * NLOC
            pltpu.sync_copy(acc.at[pl.ds(0, last)],
                            agg_hbm.at[pl.ds((NW - 1) * NLOC, last)])

    pl.run_scoped(acc_scope, pltpu.VMEM((NLOC + 1, D), jnp.float32))


def _segmin(h, lists, cnts):
    f = pl.kernel(
        _segmin_kernel,
        out_type=jax.ShapeDtypeStruct((N_NODES, D), jnp.float32),
        mesh=_mesh(),
        compiler_params=pltpu.CompilerParams(needs_layout_passes=False),
        scratch_types=[
            pltpu.VMEM((BATCH,), jnp.int32),
            pltpu.VMEM((BATCH,), jnp.int32),
            pltpu.VMEM((BATCH,), jnp.int32),
            pltpu.VMEM((BATCH,), jnp.int32),
            pltpu.VMEM((BATCH, D), jnp.float32),
            pltpu.VMEM((BATCH, D), jnp.float32),
            pltpu.VMEM((QN, 16), jnp.int32),
            pltpu.SemaphoreType.DMA,
            pltpu.SemaphoreType.DMA,
            pltpu.SemaphoreType.DMA,
            pltpu.SemaphoreType.DMA,
        ],
    )
    return f(h, lists, cnts)


ROWS = 1000
GRID = N_NODES // ROWS


def _mm_bn_kernel(agg_ref, x_ref, wl_ref, wr_ref, b_ref, h_ref, sums_ref,
                  acc_ref):
    i = pl.program_id(0)
    h = (jnp.dot(agg_ref[...], wl_ref[...], preferred_element_type=jnp.float32)
         + jnp.dot(x_ref[...], wr_ref[...], preferred_element_type=jnp.float32)
         + b_ref[...])
    h_ref[...] = h

    @pl.when(i == 0)
    def _():
        acc_ref[...] = jnp.zeros_like(acc_ref)

    s1 = jnp.sum(h, axis=0)[None, :]
    s2 = jnp.sum(h * h, axis=0)[None, :]
    acc_ref[0:1, :] += s1
    acc_ref[1:2, :] += s2

    @pl.when(i == GRID - 1)
    def _():
        sums_ref[...] = acc_ref[...]


def _mm_bn(agg, x, wl, wr, b):
    return pl.pallas_call(
        _mm_bn_kernel,
        grid=(GRID,),
        in_specs=[
            pl.BlockSpec((ROWS, D), lambda i: (i, 0)),
            pl.BlockSpec((ROWS, D), lambda i: (i, 0)),
            pl.BlockSpec((D, D), lambda i: (0, 0)),
            pl.BlockSpec((D, D), lambda i: (0, 0)),
            pl.BlockSpec((1, D), lambda i: (0, 0)),
        ],
        out_specs=[
            pl.BlockSpec((ROWS, D), lambda i: (i, 0)),
            pl.BlockSpec((8, D), lambda i: (0, 0)),
        ],
        out_shape=[
            jax.ShapeDtypeStruct((N_NODES, D), jnp.float32),
            jax.ShapeDtypeStruct((8, D), jnp.float32),
        ],
        scratch_shapes=[pltpu.VMEM((8, D), jnp.float32)],
    )(agg, x, wl, wr, b.reshape(1, -1))


def _norm_relu_kernel(h_ref, a_ref, c_ref, o_ref):
    o_ref[...] = jnp.maximum(h_ref[...] * a_ref[...] + c_ref[...], 0.0)


def _norm_relu(h, a, c):
    return pl.pallas_call(
        _norm_relu_kernel,
        grid=(GRID,),
        in_specs=[
            pl.BlockSpec((ROWS, D), lambda i: (i, 0)),
            pl.BlockSpec((1, D), lambda i: (0, 0)),
            pl.BlockSpec((1, D), lambda i: (0, 0)),
        ],
        out_specs=pl.BlockSpec((ROWS, D), lambda i: (i, 0)),
        out_shape=jax.ShapeDtypeStruct((N_NODES, D), jnp.float32),
    )(h, a.reshape(1, -1), c.reshape(1, -1))


def _final_kernel(agg_ref, x_ref, wl_ref, wr_ref, b_ref, o_ref):
    z = (jnp.dot(agg_ref[...], wl_ref[...], preferred_element_type=jnp.float32)
         + jnp.dot(x_ref[...], wr_ref[...], preferred_element_type=jnp.float32)
         + b_ref[...])
    mx = jnp.max(z, axis=1, keepdims=True)
    lse = jnp.log(jnp.sum(jnp.exp(z - mx), axis=1, keepdims=True)) + mx
    o_ref[...] = z - lse


def _final(agg, x, wl, wr, b):
    return pl.pallas_call(
        _final_kernel,
        grid=(GRID,),
        in_specs=[
            pl.BlockSpec((ROWS, D), lambda i: (i, 0)),
            pl.BlockSpec((ROWS, D), lambda i: (i, 0)),
            pl.BlockSpec((D, D_OUT), lambda i: (0, 0)),
            pl.BlockSpec((D, D_OUT), lambda i: (0, 0)),
            pl.BlockSpec((1, D_OUT), lambda i: (0, 0)),
        ],
        out_specs=pl.BlockSpec((ROWS, D_OUT), lambda i: (i, 0)),
        out_shape=jax.ShapeDtypeStruct((N_NODES, D_OUT), jnp.float32),
    )(agg, x, wl, wr, b.reshape(1, -1))


def kernel(x, edge_index, W1l, b1, W1r, g1, be1, W2l, b2, W2r, g2, be2,
           W3l, b3, W3r):
    lists, cnts = _bin_edges(edge_index)

    def sage_bn_layer(h_in, wl, b, wr, g, be):
        agg = _segmin(h_in, lists, cnts)
        h, sums = _mm_bn(agg, h_in, wl, wr, b)
        m = sums[0] / N_NODES
        var = sums[1] / N_NODES - m * m
        a = g / jnp.sqrt(var + EPS)
        c = be - m * a
        return _norm_relu(h, a, c)

    h1 = sage_bn_layer(x, W1l, b1, W1r, g1, be1)
    h2 = sage_bn_layer(h1, W2l, b2, W2r, g2, be2)
    agg3 = _segmin(h2, lists, cnts)
    return _final(agg3, h2, W3l, W3r, b3)


# speedup vs baseline: 1.1013x; 1.0085x over previous
import functools

import jax
import jax.numpy as jnp
from jax import lax
from jax.experimental import pallas as pl
from jax.experimental.pallas import tpu as pltpu
from jax.experimental.pallas import tpu_sc as plsc

N_NODES = 10000
N_EDGES = 320000
D = 128
D_OUT = 64
EPS = 1e-5

NC = 2
NS = 16
NW = NC * NS
NLOC = 320
DUMMY = NLOC

QN = 4
EQUART = N_EDGES // QN
CHUNK = 16000
NCHUNK = EQUART // CHUNK
BATCH = 128
FLUSH = 1024
RING = 4096
LCAP4 = 81920
LCAP = QN * LCAP4
BIG = 3.4028235e38


def _mesh():
    return plsc.VectorSubcoreMesh(core_axis_name="c", subcore_axis_name="s")


def _wid():
    return lax.axis_index("c") * NS + lax.axis_index("s")


def _bin_kernel(edge_hbm, lists_hbm, cnts_hbm,
                sa0, da0, sa1, da1,
                sr0, dr0, sr1, dr1, sr2, dr2, sr3, dr3,
                cnts_v, sem0, sem1):
    w = _wid()
    g = w // QN
    q = w % QN
    glo = g * QN * NLOC
    eq0 = q * EQUART
    lanes = lax.iota(jnp.int32, 16)
    srings = [sr0, sr1, sr2, sr3]
    drings = [dr0, dr1, dr2, dr3]

    def fire_stage(chunk, sv, dv, sem):
        off = eq0 + chunk * CHUNK
        pltpu.async_copy(edge_hbm.at[0, pl.ds(off, CHUNK)], sv, sem)
        pltpu.async_copy(edge_hbm.at[1, pl.ds(off, CHUNK)], dv, sem)

    def wait_stage(sv, dv, sem):
        pltpu.make_async_copy(edge_hbm.at[0, pl.ds(0, CHUNK)], sv, sem).wait()
        pltpu.make_async_copy(edge_hbm.at[1, pl.ds(0, CHUNK)], dv, sem).wait()

    def flush_one(r, fl):
        fo = pl.multiple_of(fl & (RING - 1), FLUSH)
        fh = pl.multiple_of(fl, FLUSH)
        hb = pl.multiple_of(q * LCAP4 + fh, FLUSH)
        pltpu.sync_copy(srings[r].at[pl.ds(fo, FLUSH)],
                        lists_hbm.at[g * QN + r, 0, pl.ds(hb, FLUSH)])
        pltpu.sync_copy(drings[r].at[pl.ds(fo, FLUSH)],
                        lists_hbm.at[g * QN + r, 1, pl.ds(hb, FLUSH)])
        return fl + FLUSH

    def scan_chunk(src_v, dst_v, carry):
        nsub = CHUNK // 16 // 25

        def scan_step(i, cnts):
            d = dst_v[pl.ds(i * 16, 16)]
            s = src_v[pl.ds(i * 16, 16)]
            local = d - glo
            m = (local >= 0) & (local < QN * NLOC)
            out = []
            for r in range(QN):
                mr = m & (local >= r * NLOC) & (local < (r + 1) * NLOC)
                npop = plsc.all_reduce_population_count(mr)[0]
                posm = cnts[r] & (RING - 1)
                plsc.store_compressed(drings[r].at[pl.ds(posm, 16)],
                                      local - r * NLOC, mask=mr)
                plsc.store_compressed(srings[r].at[pl.ds(posm, 16)],
                                      s, mask=mr)

                @pl.when(posm + npop > RING)
                def _(r=r):
                    drings[r][pl.ds(0, 16)] = drings[r][pl.ds(RING, 16)]
                    srings[r][pl.ds(0, 16)] = srings[r][pl.ds(RING, 16)]

                out.append(cnts[r] + npop)
            return tuple(out)

        def outer_step(o, carry):
            cnts = carry[:QN]
            fls = carry[QN:]
            cnts = lax.fori_loop(o * 25, o * 25 + 25, scan_step, cnts)
            nfls = []
            for r in range(QN):
                fl = lax.cond(cnts[r] - fls[r] >= FLUSH,
                              lambda fl, r=r: flush_one(r, fl),
                              lambda fl: fl, fls[r])
                nfls.append(fl)
            return cnts + tuple(nfls)

        return lax.fori_loop(0, nsub, outer_step, carry)

    fire_stage(0, sa0, da0, sem0)
    carry = tuple(jnp.int32(0) for _ in range(2 * QN))

    def two_chunks(t, carry):
        wait_stage(sa0, da0, sem0)
        fire_stage(2 * t + 1, sa1, da1, sem1)
        carry = scan_chunk(sa0, da0, carry)
        wait_stage(sa1, da1, sem1)

        @pl.when(2 * t + 2 < NCHUNK)
        def _():
            fire_stage(2 * t + 2, sa0, da0, sem0)

        return scan_chunk(sa1, da1, carry)

    carry = lax.fori_loop(0, NCHUNK // 2, two_chunks, carry)
    if NCHUNK % 2:
        wait_stage(sa0, da0, sem0)
        carry = scan_chunk(sa0, da0, carry)

    for r in range(QN):
        cnt = carry[r]
        flushed = carry[QN + r]
        for j in range(BATCH // 16):
            pos = (cnt + j * 16 + lanes) & (RING - 1)
            plsc.store_scatter(drings[r], [pos],
                               jnp.full((16,), DUMMY, jnp.int32))
            plsc.store_scatter(srings[r], [pos], jnp.zeros((16,), jnp.int32))

        padded = ((cnt + BATCH - 1) // BATCH) * BATCH
        ntail = (padded - flushed) // BATCH

        def tail_body(k, fl, r=r):
            fo = pl.multiple_of(fl & (RING - 1), BATCH)
            fh = pl.multiple_of(fl, BATCH)
            hb = pl.multiple_of(q * LCAP4 + fh, BATCH)
            pltpu.sync_copy(srings[r].at[pl.ds(fo, BATCH)],
                            lists_hbm.at[g * QN + r, 0, pl.ds(hb, BATCH)])
            pltpu.sync_copy(drings[r].at[pl.ds(fo, BATCH)],
                            lists_hbm.at[g * QN + r, 1, pl.ds(hb, BATCH)])
            return fl + BATCH

        lax.fori_loop(0, ntail, tail_body, flushed)

        plsc.store_scatter(cnts_v, [jnp.zeros((16,), jnp.int32)],
                           cnt + jnp.zeros((16,), jnp.int32),
                           mask=lanes == 0)
        pltpu.sync_copy(cnts_v, cnts_hbm.at[g * QN + r, q])


def _bin_edges(edge_index):
    f = pl.kernel(
        _bin_kernel,
        out_type=(
            jax.ShapeDtypeStruct((NW, 2, LCAP), jnp.int32),
            jax.ShapeDtypeStruct((NW, QN, 16), jnp.int32),
        ),
        mesh=_mesh(),
        compiler_params=pltpu.CompilerParams(needs_layout_passes=False),
        scratch_types=(
            [pltpu.VMEM((CHUNK,), jnp.int32) for _ in range(4)]
            + [pltpu.VMEM((RING + 16,), jnp.int32) for _ in range(2 * QN)]
            + [pltpu.VMEM((16,), jnp.int32),
               pltpu.SemaphoreType.DMA,
               pltpu.SemaphoreType.DMA]
        ),
    )
    return f(edge_index)


def _segmin_kernel(h_hbm, lists_hbm, cnts_hbm, agg_hbm,
                   idx0, idx1, ldst0, ldst1, rows0, rows1, cnts_v,
                   psem0, psem1, gsem0, gsem1):
    w = _wid()

    def acc_scope(acc):
        def init_body(r, _):
            for c in range(D // 16):
                acc[r, pl.ds(c * 16, 16)] = jnp.full((16,), BIG, jnp.float32)
            return 0

        lax.fori_loop(0, NLOC + 1, init_body, 0)

        pltpu.sync_copy(cnts_hbm.at[w], cnts_v)
        subcnts = [cnts_v[qq, pl.ds(0, 16)][0] for qq in range(QN)]

        def fire_gather(idx, rows, gsem):
            pltpu.async_copy(h_hbm.at[idx], rows, gsem)

        def wait_gather(idx, rows, gsem):
            pltpu.make_async_copy(h_hbm.at[idx], rows, gsem).wait()

        def compute(ldst, rows):
            def group_body(eg, _):
                dv = ldst[pl.ds(eg * 16, 16)]
                for j in range(16):
                    dd = dv[j]
                    e = eg * 16 + j
                    for c in range(D // 16):
                        cs = pl.ds(c * 16, 16)
                        acc[dd, cs] = jnp.minimum(acc[dd, cs], rows[e, cs])
                return 0

            lax.fori_loop(0, BATCH // 16, group_body, 0)

        slots = [(idx0, ldst0, rows0, psem0, gsem0),
                 (idx1, ldst1, rows1, psem1, gsem1)]

        nbs = [(c + BATCH - 1) // BATCH for c in subcnts]
        cum1 = nbs[0]
        cum2 = cum1 + nbs[1]
        cum3 = cum2 + nbs[2]
        nb = cum3 + nbs[3]

        def batch_base(b):
            qq = ((b >= cum1).astype(jnp.int32)
                  + (b >= cum2).astype(jnp.int32)
                  + (b >= cum3).astype(jnp.int32))
            start = jnp.where(qq == 1, cum1,
                              jnp.where(qq == 2, cum2,
                                        jnp.where(qq == 3, cum3, 0)))
            return pl.multiple_of(qq * LCAP4 + (b - start) * BATCH, BATCH)

        def fire_idx(b, idx, ldst, psem):
            base = batch_base(b)
            pltpu.async_copy(lists_hbm.at[w, 0, pl.ds(base, BATCH)],
                             idx, psem)
            pltpu.async_copy(lists_hbm.at[w, 1, pl.ds(base, BATCH)],
                             ldst, psem)

        def wait_idx(idx, ldst, psem):
            pltpu.make_async_copy(lists_hbm.at[w, 0, pl.ds(0, BATCH)],
                                  idx, psem).wait()
            pltpu.make_async_copy(lists_hbm.at[w, 1, pl.ds(0, BATCH)],
                                  ldst, psem).wait()

        @pl.when(nb > 0)
        def _():
            fire_idx(0, idx0, ldst0, psem0)

            @pl.when(1 < nb)
            def _():
                fire_idx(1, idx1, ldst1, psem1)

            wait_idx(idx0, ldst0, psem0)
            fire_gather(idx0, rows0, gsem0)

            def pair_body(u, _):
                for s in range(2):
                    iA, dA, rA, pA, gA = slots[s]
                    iB, dB, rB, pB, gB = slots[1 - s]
                    b = 2 * u + s

                    @pl.when(b < nb)
                    def _(b=b, iA=iA, dA=dA, rA=rA, pA=pA, gA=gA,
                          iB=iB, dB=dB, rB=rB, pB=pB, gB=gB):
                        wait_gather(iA, rA, gA)

                        @pl.when(b + 1 < nb)
                        def _():
                            wait_idx(iB, dB, pB)
                            fire_gather(iB, rB, gB)

                        compute(dA, rA)

                        @pl.when(b + 2 < nb)
                        def _():
                            fire_idx(b + 2, iA, dA, pA)

                return 0

            lax.fori_loop(0, (nb + 1) // 2, pair_body, 0)

        def fix_body(r, _):
            for c in range(D // 16):
                cs = pl.ds(c * 16, 16)
                v = acc[r, cs]
                acc[r, cs] = jnp.where(v > 3e38, jnp.float32(0.0), v)
            return 0

        lax.fori_loop(0, NLOC, fix_body, 0)

        @pl.when(w < NW - 1)
        def _():
            pltpu.sync_copy(acc.at[pl.ds(0, NLOC)],
                            agg_hbm.at[pl.ds(w * NLOC, NLOC)])

        @pl.when(w == NW - 1)
        def _():
            last = N_NODES - (NW - 1) * NLOC
            pltpu.sync_copy(acc.at[pl.ds(0, last)],
                            agg_hbm.at[pl.ds((NW - 1) * NLOC, last)])

    pl.run_scoped(acc_scope, pltpu.VMEM((NLOC + 1, D), jnp.float32))


def _segmin(h, lists, cnts):
    f = pl.kernel(
        _segmin_kernel,
        out_type=jax.ShapeDtypeStruct((N_NODES, D), jnp.float32),
        mesh=_mesh(),
        compiler_params=pltpu.CompilerParams(needs_layout_passes=False),
        scratch_types=[
            pltpu.VMEM((BATCH,), jnp.int32),
            pltpu.VMEM((BATCH,), jnp.int32),
            pltpu.VMEM((BATCH,), jnp.int32),
            pltpu.VMEM((BATCH,), jnp.int32),
            pltpu.VMEM((BATCH, D), jnp.float32),
            pltpu.VMEM((BATCH, D), jnp.float32),
            pltpu.VMEM((QN, 16), jnp.int32),
            pltpu.SemaphoreType.DMA,
            pltpu.SemaphoreType.DMA,
            pltpu.SemaphoreType.DMA,
            pltpu.SemaphoreType.DMA,
        ],
    )
    return f(h, lists, cnts)


ROWS = 1000
GRID = N_NODES // ROWS


def _mm_bn_kernel(agg_ref, x_ref, wl_ref, wr_ref, b_ref, h_ref, sums_ref,
                  acc_ref):
    i = pl.program_id(0)
    h = (jnp.dot(agg_ref[...], wl_ref[...], preferred_element_type=jnp.float32)
         + jnp.dot(x_ref[...], wr_ref[...], preferred_element_type=jnp.float32)
         + b_ref[...])
    h_ref[...] = h

    @pl.when(i == 0)
    def _():
        acc_ref[...] = jnp.zeros_like(acc_ref)

    s1 = jnp.sum(h, axis=0)[None, :]
    s2 = jnp.sum(h * h, axis=0)[None, :]
    acc_ref[0:1, :] += s1
    acc_ref[1:2, :] += s2

    @pl.when(i == GRID - 1)
    def _():
        sums_ref[...] = acc_ref[...]


def _mm_bn(agg, x, wl, wr, b):
    return pl.pallas_call(
        _mm_bn_kernel,
        grid=(GRID,),
        in_specs=[
            pl.BlockSpec((ROWS, D), lambda i: (i, 0)),
            pl.BlockSpec((ROWS, D), lambda i: (i, 0)),
            pl.BlockSpec((D, D), lambda i: (0, 0)),
            pl.BlockSpec((D, D), lambda i: (0, 0)),
            pl.BlockSpec((1, D), lambda i: (0, 0)),
        ],
        out_specs=[
            pl.BlockSpec((ROWS, D), lambda i: (i, 0)),
            pl.BlockSpec((8, D), lambda i: (0, 0)),
        ],
        out_shape=[
            jax.ShapeDtypeStruct((N_NODES, D), jnp.float32),
            jax.ShapeDtypeStruct((8, D), jnp.float32),
        ],
        scratch_shapes=[pltpu.VMEM((8, D), jnp.float32)],
    )(agg, x, wl, wr, b.reshape(1, -1))


def _norm_relu_kernel(h_ref, a_ref, c_ref, o_ref):
    o_ref[...] = jnp.maximum(h_ref[...] * a_ref[...] + c_ref[...], 0.0)


def _norm_relu(h, a, c):
    return pl.pallas_call(
        _norm_relu_kernel,
        grid=(GRID,),
        in_specs=[
            pl.BlockSpec((ROWS, D), lambda i: (i, 0)),
            pl.BlockSpec((1, D), lambda i: (0, 0)),
            pl.BlockSpec((1, D), lambda i: (0, 0)),
        ],
        out_specs=pl.BlockSpec((ROWS, D), lambda i: (i, 0)),
        out_shape=jax.ShapeDtypeStruct((N_NODES, D), jnp.float32),
    )(h, a.reshape(1, -1), c.reshape(1, -1))


def _final_kernel(agg_ref, x_ref, wl_ref, wr_ref, b_ref, o_ref):
    z = (jnp.dot(agg_ref[...], wl_ref[...], preferred_element_type=jnp.float32)
         + jnp.dot(x_ref[...], wr_ref[...], preferred_element_type=jnp.float32)
         + b_ref[...])
    mx = jnp.max(z, axis=1, keepdims=True)
    lse = jnp.log(jnp.sum(jnp.exp(z - mx), axis=1, keepdims=True)) + mx
    o_ref[...] = z - lse


def _final(agg, x, wl, wr, b):
    return pl.pallas_call(
        _final_kernel,
        grid=(GRID,),
        in_specs=[
            pl.BlockSpec((ROWS, D), lambda i: (i, 0)),
            pl.BlockSpec((ROWS, D), lambda i: (i, 0)),
            pl.BlockSpec((D, D_OUT), lambda i: (0, 0)),
            pl.BlockSpec((D, D_OUT), lambda i: (0, 0)),
            pl.BlockSpec((1, D_OUT), lambda i: (0, 0)),
        ],
        out_specs=pl.BlockSpec((ROWS, D_OUT), lambda i: (i, 0)),
        out_shape=jax.ShapeDtypeStruct((N_NODES, D_OUT), jnp.float32),
    )(agg, x, wl, wr, b.reshape(1, -1))


def kernel(x, edge_index, W1l, b1, W1r, g1, be1, W2l, b2, W2r, g2, be2,
           W3l, b3, W3r):
    lists, cnts = _bin_edges(edge_index)

    def sage_bn_layer(h_in, wl, b, wr, g, be):
        agg = _segmin(h_in, lists, cnts)
        h, sums = _mm_bn(agg, h_in, wl, wr, b)
        m = sums[0] / N_NODES
        var = sums[1] / N_NODES - m * m
        a = g / jnp.sqrt(var + EPS)
        c = be - m * a
        return _norm_relu(h, a, c)

    h1 = sage_bn_layer(x, W1l, b1, W1r, g1, be1)
    h2 = sage_bn_layer(h1, W2l, b2, W2r, g2, be2)
    agg3 = _segmin(h2, lists, cnts)
    return _final(agg3, h2, W3l, W3r, b3)


# speedup vs baseline: 1.1314x; 1.0273x over previous
import functools

import jax
import jax.numpy as jnp
from jax import lax
from jax.experimental import pallas as pl
from jax.experimental.pallas import tpu as pltpu
from jax.experimental.pallas import tpu_sc as plsc

N_NODES = 10000
N_EDGES = 320000
D = 128
D_OUT = 64
EPS = 1e-5

NC = 2
NS = 16
NW = NC * NS
NLOC = 320
DUMMY = NLOC

QN = 4
EQUART = N_EDGES // QN
CHUNK = 16000
NCHUNK = EQUART // CHUNK
BATCH = 128
FLUSH = 1024
RING = 4096
LCAP4 = 81920
LCAP = QN * LCAP4
BIG = 3.4028235e38


def _mesh():
    return plsc.VectorSubcoreMesh(core_axis_name="c", subcore_axis_name="s")


def _wid():
    return lax.axis_index("c") * NS + lax.axis_index("s")


def _bin_kernel(edge_hbm, lists_hbm, cnts_hbm,
                sa0, da0, sa1, da1,
                sr0, dr0, sr1, dr1, sr2, dr2, sr3, dr3,
                cnts_v, sem0, sem1):
    w = _wid()
    g = w // QN
    q = w % QN
    glo = g * QN * NLOC
    eq0 = q * EQUART
    lanes = lax.iota(jnp.int32, 16)
    srings = [sr0, sr1, sr2, sr3]
    drings = [dr0, dr1, dr2, dr3]

    def fire_stage(chunk, sv, dv, sem):
        off = eq0 + chunk * CHUNK
        pltpu.async_copy(edge_hbm.at[0, pl.ds(off, CHUNK)], sv, sem)
        pltpu.async_copy(edge_hbm.at[1, pl.ds(off, CHUNK)], dv, sem)

    def wait_stage(sv, dv, sem):
        pltpu.make_async_copy(edge_hbm.at[0, pl.ds(0, CHUNK)], sv, sem).wait()
        pltpu.make_async_copy(edge_hbm.at[1, pl.ds(0, CHUNK)], dv, sem).wait()

    def flush_one(r, fl):
        fo = pl.multiple_of(fl & (RING - 1), FLUSH)
        fh = pl.multiple_of(fl, FLUSH)
        hb = pl.multiple_of(q * LCAP4 + fh, FLUSH)
        pltpu.sync_copy(srings[r].at[pl.ds(fo, FLUSH)],
                        lists_hbm.at[g * QN + r, 0, pl.ds(hb, FLUSH)])
        pltpu.sync_copy(drings[r].at[pl.ds(fo, FLUSH)],
                        lists_hbm.at[g * QN + r, 1, pl.ds(hb, FLUSH)])
        return fl + FLUSH

    def scan_chunk(src_v, dst_v, carry):
        nsub = CHUNK // 16 // 25

        def scan_step(i, cnts):
            d = dst_v[pl.ds(i * 16, 16)]
            s = src_v[pl.ds(i * 16, 16)]
            local = d - glo
            m = (local >= 0) & (local < QN * NLOC)
            out = []
            for r in range(QN):
                mr = m & (local >= r * NLOC) & (local < (r + 1) * NLOC)
                npop = plsc.all_reduce_population_count(mr)[0]
                posm = cnts[r] & (RING - 1)
                plsc.store_compressed(drings[r].at[pl.ds(posm, 16)],
                                      local - r * NLOC, mask=mr)
                plsc.store_compressed(srings[r].at[pl.ds(posm, 16)],
                                      s, mask=mr)

                @pl.when(posm + npop > RING)
                def _(r=r):
                    drings[r][pl.ds(0, 16)] = drings[r][pl.ds(RING, 16)]
                    srings[r][pl.ds(0, 16)] = srings[r][pl.ds(RING, 16)]

                out.append(cnts[r] + npop)
            return tuple(out)

        def outer_step(o, carry):
            cnts = carry[:QN]
            fls = carry[QN:]
            cnts = lax.fori_loop(o * 25, o * 25 + 25, scan_step, cnts)
            nfls = []
            for r in range(QN):
                fl = lax.cond(cnts[r] - fls[r] >= FLUSH,
                              lambda fl, r=r: flush_one(r, fl),
                              lambda fl: fl, fls[r])
                nfls.append(fl)
            return cnts + tuple(nfls)

        return lax.fori_loop(0, nsub, outer_step, carry)

    fire_stage(0, sa0, da0, sem0)
    carry = tuple(jnp.int32(0) for _ in range(2 * QN))

    def two_chunks(t, carry):
        wait_stage(sa0, da0, sem0)
        fire_stage(2 * t + 1, sa1, da1, sem1)
        carry = scan_chunk(sa0, da0, carry)
        wait_stage(sa1, da1, sem1)

        @pl.when(2 * t + 2 < NCHUNK)
        def _():
            fire_stage(2 * t + 2, sa0, da0, sem0)

        return scan_chunk(sa1, da1, carry)

    carry = lax.fori_loop(0, NCHUNK // 2, two_chunks, carry)
    if NCHUNK % 2:
        wait_stage(sa0, da0, sem0)
        carry = scan_chunk(sa0, da0, carry)

    for r in range(QN):
        cnt = carry[r]
        flushed = carry[QN + r]
        for j in range(BATCH // 16):
            pos = (cnt + j * 16 + lanes) & (RING - 1)
            plsc.store_scatter(drings[r], [pos],
                               jnp.full((16,), DUMMY, jnp.int32))
            plsc.store_scatter(srings[r], [pos], jnp.zeros((16,), jnp.int32))

        padded = ((cnt + BATCH - 1) // BATCH) * BATCH
        ntail = (padded - flushed) // BATCH

        def tail_body(k, fl, r=r):
            fo = pl.multiple_of(fl & (RING - 1), BATCH)
            fh = pl.multiple_of(fl, BATCH)
            hb = pl.multiple_of(q * LCAP4 + fh, BATCH)
            pltpu.sync_copy(srings[r].at[pl.ds(fo, BATCH)],
                            lists_hbm.at[g * QN + r, 0, pl.ds(hb, BATCH)])
            pltpu.sync_copy(drings[r].at[pl.ds(fo, BATCH)],
                            lists_hbm.at[g * QN + r, 1, pl.ds(hb, BATCH)])
            return fl + BATCH

        lax.fori_loop(0, ntail, tail_body, flushed)

        plsc.store_scatter(cnts_v, [jnp.zeros((16,), jnp.int32)],
                           cnt + jnp.zeros((16,), jnp.int32),
                           mask=lanes == 0)
        pltpu.sync_copy(cnts_v, cnts_hbm.at[g * QN + r, q])


def _bin_edges(edge_index):
    f = pl.kernel(
        _bin_kernel,
        out_type=(
            jax.ShapeDtypeStruct((NW, 2, LCAP), jnp.int32),
            jax.ShapeDtypeStruct((NW, QN, 16), jnp.int32),
        ),
        mesh=_mesh(),
        compiler_params=pltpu.CompilerParams(needs_layout_passes=False),
        scratch_types=(
            [pltpu.VMEM((CHUNK,), jnp.int32) for _ in range(4)]
            + [pltpu.VMEM((RING + 16,), jnp.int32) for _ in range(2 * QN)]
            + [pltpu.VMEM((16,), jnp.int32),
               pltpu.SemaphoreType.DMA,
               pltpu.SemaphoreType.DMA]
        ),
    )
    return f(edge_index)


GRP = 16
DEPTH = 4


def _segmin_kernel(h_hbm, lists_hbm, cnts_hbm, agg_hbm,
                   pb, rows0, rows1, rows2, rows3, cnts_v,
                   gsem0, gsem1, gsem2, gsem3):
    w = _wid()

    def acc_scope(acc):
        def init_body(r, _):
            for c in range(D // 16):
                acc[r, pl.ds(c * 16, 16)] = jnp.full((16,), BIG, jnp.float32)
            return 0

        lax.fori_loop(0, NLOC + 1, init_body, 0)

        pltpu.sync_copy(cnts_hbm.at[w], cnts_v)
        subcnts = [cnts_v[qq, pl.ds(0, 16)][0] for qq in range(QN)]

        def fire(off, rows, gsem):
            pltpu.async_copy(h_hbm.at[pb.at[0, pl.ds(off, BATCH)]],
                             rows, gsem)

        def wait(off, rows, gsem):
            pltpu.make_async_copy(h_hbm.at[pb.at[0, pl.ds(off, BATCH)]],
                                  rows, gsem).wait()

        def compute(base, rows):
            def group_body(eg, _):
                dv = pb[1, pl.ds(base + eg * 16, 16)]
                for j in range(16):
                    dd = dv[j]
                    e = eg * 16 + j
                    for c in range(D // 16):
                        cs = pl.ds(c * 16, 16)
                        acc[dd, cs] = jnp.minimum(acc[dd, cs], rows[e, cs])
                return 0

            lax.fori_loop(0, BATCH // 16, group_body, 0)

        rows = [rows0, rows1, rows2, rows3]
        gsems = [gsem0, gsem1, gsem2, gsem3]

        def seg_body(qq, _):
            cnt = subcnts[0]
            for q2 in range(1, QN):
                cnt = jnp.where(qq == q2, subcnts[q2], cnt)
            nb = (cnt + BATCH - 1) // BATCH
            base0 = qq * LCAP4
            ngrp = (nb + GRP - 1) // GRP

            def grp_body(sg, _):
                gb = pl.multiple_of(base0 + sg * (GRP * BATCH), GRP * BATCH)
                pltpu.sync_copy(lists_hbm.at[w, :, pl.ds(gb, GRP * BATCH)],
                                pb)
                b0 = sg * GRP
                bg = jnp.minimum(nb - b0, GRP)

                fire(0, rows[0], gsems[0])
                for s in range(1, DEPTH - 1):
                    @pl.when(b0 + s < nb)
                    def _(s=s):
                        fire(s * BATCH, rows[s], gsems[s])

                def quad_body(u, _):
                    q0 = u * DEPTH
                    for s in range(DEPTH):
                        k = q0 + s
                        kf = k + DEPTH - 1
                        off = pl.multiple_of(k * BATCH, BATCH)
                        off_f = pl.multiple_of(kf * BATCH, BATCH)
                        sf = (s + DEPTH - 1) % DEPTH

                        @pl.when(jnp.logical_and(kf < GRP, b0 + kf < nb))
                        def _(off_f=off_f, sf=sf):
                            fire(off_f, rows[sf], gsems[sf])

                        @pl.when(b0 + k < nb)
                        def _(off=off, s=s):
                            wait(off, rows[s], gsems[s])
                            compute(off, rows[s])
                    return 0

                nquad = (bg + DEPTH - 1) // DEPTH
                lax.fori_loop(0, nquad, quad_body, 0)
                return 0

            lax.fori_loop(0, ngrp, grp_body, 0)
            return 0

        lax.fori_loop(0, QN, seg_body, 0)

        def fix_body(r, _):
            for c in range(D // 16):
                cs = pl.ds(c * 16, 16)
                v = acc[r, cs]
                acc[r, cs] = jnp.where(v > 3e38, jnp.float32(0.0), v)
            return 0

        lax.fori_loop(0, NLOC, fix_body, 0)

        @pl.when(w < NW - 1)
        def _():
            pltpu.sync_copy(acc.at[pl.ds(0, NLOC)],
                            agg_hbm.at[pl.ds(w * NLOC, NLOC)])

        @pl.when(w == NW - 1)
        def _():
            last = N_NODES - (NW - 1) * NLOC
            pltpu.sync_copy(acc.at[pl.ds(0, last)],
                            agg_hbm.at[pl.ds((NW - 1) * NLOC, last)])

    pl.run_scoped(acc_scope, pltpu.VMEM((NLOC + 1, D), jnp.float32))


def _segmin(h, lists, cnts):
    f = pl.kernel(
        _segmin_kernel,
        out_type=jax.ShapeDtypeStruct((N_NODES, D), jnp.float32),
        mesh=_mesh(),
        compiler_params=pltpu.CompilerParams(needs_layout_passes=False),
        scratch_types=[
            pltpu.VMEM((2, GRP * BATCH), jnp.int32),
            pltpu.VMEM((BATCH, D), jnp.float32),
            pltpu.VMEM((BATCH, D), jnp.float32),
            pltpu.VMEM((BATCH, D), jnp.float32),
            pltpu.VMEM((BATCH, D), jnp.float32),
            pltpu.VMEM((QN, 16), jnp.int32),
            pltpu.SemaphoreType.DMA,
            pltpu.SemaphoreType.DMA,
            pltpu.SemaphoreType.DMA,
            pltpu.SemaphoreType.DMA,
        ],
    )
    return f(h, lists, cnts)


ROWS = 1000
GRID = N_NODES // ROWS


def _mm_bn_kernel(agg_ref, x_ref, wl_ref, wr_ref, b_ref, h_ref, sums_ref,
                  acc_ref):
    i = pl.program_id(0)
    h = (jnp.dot(agg_ref[...], wl_ref[...], preferred_element_type=jnp.float32)
         + jnp.dot(x_ref[...], wr_ref[...], preferred_element_type=jnp.float32)
         + b_ref[...])
    h_ref[...] = h

    @pl.when(i == 0)
    def _():
        acc_ref[...] = jnp.zeros_like(acc_ref)

    s1 = jnp.sum(h, axis=0)[None, :]
    s2 = jnp.sum(h * h, axis=0)[None, :]
    acc_ref[0:1, :] += s1
    acc_ref[1:2, :] += s2

    @pl.when(i == GRID - 1)
    def _():
        sums_ref[...] = acc_ref[...]


def _mm_bn(agg, x, wl, wr, b):
    return pl.pallas_call(
        _mm_bn_kernel,
        grid=(GRID,),
        in_specs=[
            pl.BlockSpec((ROWS, D), lambda i: (i, 0)),
            pl.BlockSpec((ROWS, D), lambda i: (i, 0)),
            pl.BlockSpec((D, D), lambda i: (0, 0)),
            pl.BlockSpec((D, D), lambda i: (0, 0)),
            pl.BlockSpec((1, D), lambda i: (0, 0)),
        ],
        out_specs=[
            pl.BlockSpec((ROWS, D), lambda i: (i, 0)),
            pl.BlockSpec((8, D), lambda i: (0, 0)),
        ],
        out_shape=[
            jax.ShapeDtypeStruct((N_NODES, D), jnp.float32),
            jax.ShapeDtypeStruct((8, D), jnp.float32),
        ],
        scratch_shapes=[pltpu.VMEM((8, D), jnp.float32)],
    )(agg, x, wl, wr, b.reshape(1, -1))


def _norm_relu_kernel(h_ref, a_ref, c_ref, o_ref):
    o_ref[...] = jnp.maximum(h_ref[...] * a_ref[...] + c_ref[...], 0.0)


def _norm_relu(h, a, c):
    return pl.pallas_call(
        _norm_relu_kernel,
        grid=(GRID,),
        in_specs=[
            pl.BlockSpec((ROWS, D), lambda i: (i, 0)),
            pl.BlockSpec((1, D), lambda i: (0, 0)),
            pl.BlockSpec((1, D), lambda i: (0, 0)),
        ],
        out_specs=pl.BlockSpec((ROWS, D), lambda i: (i, 0)),
        out_shape=jax.ShapeDtypeStruct((N_NODES, D), jnp.float32),
    )(h, a.reshape(1, -1), c.reshape(1, -1))


def _final_kernel(agg_ref, x_ref, wl_ref, wr_ref, b_ref, o_ref):
    z = (jnp.dot(agg_ref[...], wl_ref[...], preferred_element_type=jnp.float32)
         + jnp.dot(x_ref[...], wr_ref[...], preferred_element_type=jnp.float32)
         + b_ref[...])
    mx = jnp.max(z, axis=1, keepdims=True)
    lse = jnp.log(jnp.sum(jnp.exp(z - mx), axis=1, keepdims=True)) + mx
    o_ref[...] = z - lse


def _final(agg, x, wl, wr, b):
    return pl.pallas_call(
        _final_kernel,
        grid=(GRID,),
        in_specs=[
            pl.BlockSpec((ROWS, D), lambda i: (i, 0)),
            pl.BlockSpec((ROWS, D), lambda i: (i, 0)),
            pl.BlockSpec((D, D_OUT), lambda i: (0, 0)),
            pl.BlockSpec((D, D_OUT), lambda i: (0, 0)),
            pl.BlockSpec((1, D_OUT), lambda i: (0, 0)),
        ],
        out_specs=pl.BlockSpec((ROWS, D_OUT), lambda i: (i, 0)),
        out_shape=jax.ShapeDtypeStruct((N_NODES, D_OUT), jnp.float32),
    )(agg, x, wl, wr, b.reshape(1, -1))


def kernel(x, edge_index, W1l, b1, W1r, g1, be1, W2l, b2, W2r, g2, be2,
           W3l, b3, W3r):
    lists, cnts = _bin_edges(edge_index)

    def sage_bn_layer(h_in, wl, b, wr, g, be):
        agg = _segmin(h_in, lists, cnts)
        h, sums = _mm_bn(agg, h_in, wl, wr, b)
        m = sums[0] / N_NODES
        var = sums[1] / N_NODES - m * m
        a = g / jnp.sqrt(var + EPS)
        c = be - m * a
        return _norm_relu(h, a, c)

    h1 = sage_bn_layer(x, W1l, b1, W1r, g1, be1)
    h2 = sage_bn_layer(h1, W2l, b2, W2r, g2, be2)
    agg3 = _segmin(h2, lists, cnts)
    return _final(agg3, h2, W3l, W3r, b3)
